# baseline (speedup 1.0000x reference)
import functools

import jax
import jax.numpy as jnp
from jax import lax
from jax.experimental import pallas as pl
from jax.experimental.pallas import tpu as pltpu
from jax.experimental.pallas import tpu_sc as plsc

_NC = 2
_NS = 16
_L = 16
_NP = 8
_B = _NP * 64

_mesh = plsc.VectorSubcoreMesh(core_axis_name="c", subcore_axis_name="s")


@functools.partial(
    pl.kernel,
    mesh=_mesh,
    out_type=jax.ShapeDtypeStruct((_B, 64), jnp.float32),
    scratch_types=[
        pltpu.VMEM((2 * _NP, _L), jnp.int32),
        pltpu.SemaphoreType.DMA,
    ],
)
def _gather_sc(x_hbm, pack_hbm, out_hbm, pack_v, sem):
    wid = lax.axis_index("s") * _NC + lax.axis_index("c")

    @pl.when(wid < _NP)
    def _():
        pltpu.sync_copy(pack_hbm, pack_v)
        i1 = pack_v[wid][0]
        i2 = pack_v[wid + _NP][0]
        pltpu.async_copy(
            x_hbm.at[i1, :, i2, :], out_hbm.at[pl.ds(wid * 64, 64), :], sem
        ).wait()


def kernel(x, index1, index2):
    pairs = jnp.concatenate(
        [index1.reshape(8).astype(jnp.int32), index2.reshape(8).astype(jnp.int32)]
    )
    pack = jnp.broadcast_to(pairs[:, None], (2 * _NP, _L))
    return _gather_sc(x, pack).reshape(4, 2, 64, 64)


# speedup vs baseline: 1.7446x; 1.7446x over previous
import functools

import jax
import jax.numpy as jnp
from jax import lax
from jax.experimental import pallas as pl
from jax.experimental.pallas import tpu as pltpu
from jax.experimental.pallas import tpu_sc as plsc

_NC = 2
_NS = 16
_L = 16
_NP = 8
_B = _NP * 64

_mesh = plsc.VectorSubcoreMesh(core_axis_name="c", subcore_axis_name="s")


@functools.partial(
    pl.kernel,
    mesh=_mesh,
    out_type=jax.ShapeDtypeStruct((_B, 64), jnp.float32),
    scratch_types=[
        pltpu.VMEM((2 * _NP, _L), jnp.int32),
        pltpu.SemaphoreType.DMA,
    ],
)
def _gather_sc(x_hbm, pack_hbm, out_hbm, pack_v, sem):
    wid = lax.axis_index("s") * _NC + lax.axis_index("c")

    @pl.when(wid < _NP)
    def _():
        pltpu.sync_copy(pack_hbm, pack_v)
        i1 = pack_v[wid][0]
        i2 = pack_v[wid + _NP][0]
        pltpu.async_copy(
            x_hbm.at[pl.ds(i1 * 64, 64), i2, :],
            out_hbm.at[pl.ds(wid * 64, 64), :],
            sem,
        ).wait()


def kernel(x, index1, index2):
    x3 = x.reshape(128 * 64, 128, 64)
    pairs = jnp.concatenate(
        [index1.reshape(8).astype(jnp.int32), index2.reshape(8).astype(jnp.int32)]
    )
    pack = jnp.broadcast_to(pairs[:, None], (2 * _NP, _L))
    return _gather_sc(x3, pack).reshape(4, 2, 64, 64)
